# baseline (speedup 1.0000x reference)
import jax
import jax.numpy as jnp
from jax import lax
from jax.experimental import pallas as pl
from jax.experimental.pallas import tpu as pltpu
from jax.experimental.pallas import tpu_sc as plsc

_B = 16384
_D = 64
_MU = 3.5
_NC = 2
_NS = 16
_NW = _NC * _NS
_BPW = _B // _NW
_CH = 128
_NCH = _BPW // _CH
_L = 16
_NV = _D // _L


def _sc_body(uidx_hbm, iidx_hbm, uemb_hbm, iemb_hbm, ubias_hbm, ibias_hbm,
             acc_hbm, ubg_hbm, ibg_hbm, uidx_v, iidx_v, urows_v, irows_v,
             ubias_v, ibias_v, acc_v, sem):
    wid = lax.axis_index("s") * _NC + lax.axis_index("c")
    base = wid * _BPW

    pltpu.sync_copy(uidx_hbm.at[pl.ds(wid * _NCH, _NCH)], uidx_v)
    pltpu.sync_copy(iidx_hbm.at[pl.ds(wid * _NCH, _NCH)], iidx_v)

    copies = []
    for j in range(_NCH):
        copies.append(pltpu.async_copy(
            uemb_hbm.at[uidx_v.at[j]], urows_v.at[pl.ds(j * _CH, _CH)], sem))
        copies.append(pltpu.async_copy(
            iemb_hbm.at[iidx_v.at[j]], irows_v.at[pl.ds(j * _CH, _CH)], sem))
        copies.append(pltpu.async_copy(
            ubias_hbm.at[uidx_v.at[j]], ubias_v.at[pl.ds(j * _CH, _CH)], sem))
        copies.append(pltpu.async_copy(
            ibias_hbm.at[iidx_v.at[j]], ibias_v.at[pl.ds(j * _CH, _CH)], sem))
    for cp in copies:
        cp.wait()

    def row(b, carry):
        acc = jnp.zeros((_L,), jnp.float32)
        for i in range(_NV):
            u = urows_v[b, pl.ds(i * _L, _L)]
            v = irows_v[b, pl.ds(i * _L, _L)]
            acc = acc + u * v
        acc_v[b, :] = acc
        return carry

    lax.fori_loop(0, _BPW, row, 0)

    pltpu.sync_copy(acc_v, acc_hbm.at[pl.ds(base, _BPW)])
    pltpu.sync_copy(ubias_v, ubg_hbm.at[pl.ds(base, _BPW)])
    pltpu.sync_copy(ibias_v, ibg_hbm.at[pl.ds(base, _BPW)])


def _tc_body(acc_ref, ub_ref, ib_ref, o_ref):
    o_ref[...] = (jnp.sum(acc_ref[...], axis=-1) + ub_ref[...] + ib_ref[...]
                  + (2.0 * _MU))


@jax.jit
def kernel(user_indices, item_indices, user_embedding, item_embedding,
           user_bias, item_bias):
    uidx = user_indices.astype(jnp.int32).reshape(_NW * _NCH, _CH)
    iidx = item_indices.astype(jnp.int32).reshape(_NW * _NCH, _CH)
    ub = user_bias.reshape(-1)
    ib = item_bias.reshape(-1)

    mesh = plsc.VectorSubcoreMesh(core_axis_name="c", subcore_axis_name="s")
    sc_run = pl.kernel(
        _sc_body,
        out_type=[
            jax.ShapeDtypeStruct((_B, _L), jnp.float32),
            jax.ShapeDtypeStruct((_B,), jnp.float32),
            jax.ShapeDtypeStruct((_B,), jnp.float32),
        ],
        mesh=mesh,
        compiler_params=pltpu.CompilerParams(use_tc_tiling_on_sc=False),
        scratch_types=[
            pltpu.VMEM((_NCH, _CH), jnp.int32),
            pltpu.VMEM((_NCH, _CH), jnp.int32),
            pltpu.VMEM((_BPW, _D), jnp.float32),
            pltpu.VMEM((_BPW, _D), jnp.float32),
            pltpu.VMEM((_BPW,), jnp.float32),
            pltpu.VMEM((_BPW,), jnp.float32),
            pltpu.VMEM((_BPW, _L), jnp.float32),
            pltpu.SemaphoreType.DMA,
        ],
    )
    acc, ubg, ibg = sc_run(uidx, iidx, user_embedding, item_embedding, ub, ib)

    rows_per_blk = 1024
    grid = (_B // rows_per_blk,)
    out = pl.pallas_call(
        _tc_body,
        grid=grid,
        in_specs=[
            pl.BlockSpec((rows_per_blk, _L), lambda i: (i, 0)),
            pl.BlockSpec((rows_per_blk,), lambda i: (i,)),
            pl.BlockSpec((rows_per_blk,), lambda i: (i,)),
        ],
        out_specs=pl.BlockSpec((rows_per_blk,), lambda i: (i,)),
        out_shape=jax.ShapeDtypeStruct((_B,), jnp.float32),
    )(acc, ubg, ibg)
    return out


# speedup vs baseline: 1.0036x; 1.0036x over previous
import jax
import jax.numpy as jnp
from jax import lax
from jax.experimental import pallas as pl
from jax.experimental.pallas import tpu as pltpu
from jax.experimental.pallas import tpu_sc as plsc

_B = 16384
_D = 64
_MU = 3.5
_NC = 2
_NS = 16
_NW = _NC * _NS
_BPW = _B // _NW
_CH = 128
_NCH = _BPW // _CH
_L = 16
_NV = _D // _L


def _sc_body(uidx_hbm, iidx_hbm, uemb_hbm, iemb_hbm, ubias_hbm, ibias_hbm,
             acc_hbm, ubg_hbm, ibg_hbm,
             uidx_v, iidx_v, u0_v, u1_v, i0_v, i1_v,
             ubias_v, ibias_v, acc_v, sem, bsem):
    wid = lax.axis_index("s") * _NC + lax.axis_index("c")
    base = wid * _BPW

    pltpu.sync_copy(uidx_hbm.at[pl.ds(wid * _NCH, _NCH)], uidx_v)
    pltpu.sync_copy(iidx_hbm.at[pl.ds(wid * _NCH, _NCH)], iidx_v)

    bias_copies = []
    for j in range(_NCH):
        bias_copies.append(pltpu.async_copy(
            ubias_hbm.at[uidx_v.at[j]], ubias_v.at[pl.ds(j * _CH, _CH)],
            bsem))
        bias_copies.append(pltpu.async_copy(
            ibias_hbm.at[iidx_v.at[j]], ibias_v.at[pl.ds(j * _CH, _CH)],
            bsem))

    ubufs = [u0_v, u1_v]
    ibufs = [i0_v, i1_v]

    def fire(j):
        return (pltpu.async_copy(uemb_hbm.at[uidx_v.at[j]], ubufs[j % 2],
                                 sem),
                pltpu.async_copy(iemb_hbm.at[iidx_v.at[j]], ibufs[j % 2],
                                 sem))

    inflight = fire(0)
    for j in range(_NCH):
        cu, ci = inflight
        if j + 1 < _NCH:
            nxt = fire(j + 1)
        cu.wait()
        ci.wait()
        if j + 1 < _NCH:
            inflight = nxt

        ubuf = ubufs[j % 2]
        ibuf = ibufs[j % 2]

        def row(b, carry):
            bb = j * _CH + b
            acc = jnp.zeros((_L,), jnp.float32)
            for i in range(_NV):
                u = ubuf[b, pl.ds(i * _L, _L)]
                v = ibuf[b, pl.ds(i * _L, _L)]
                acc = acc + u * v
            acc_v[bb, :] = acc
            return carry

        lax.fori_loop(0, _CH, row, 0)

    for cp in bias_copies:
        cp.wait()

    pltpu.sync_copy(acc_v, acc_hbm.at[pl.ds(base, _BPW)])
    pltpu.sync_copy(ubias_v, ubg_hbm.at[pl.ds(base, _BPW)])
    pltpu.sync_copy(ibias_v, ibg_hbm.at[pl.ds(base, _BPW)])


def _tc_body(acc_ref, ub_ref, ib_ref, o_ref):
    o_ref[...] = (jnp.sum(acc_ref[...], axis=-1) + ub_ref[...] + ib_ref[...]
                  + (2.0 * _MU))


@jax.jit
def kernel(user_indices, item_indices, user_embedding, item_embedding,
           user_bias, item_bias):
    ui = user_indices.astype(jnp.int32)
    ii = item_indices.astype(jnp.int32)
    uidx = ui.reshape(_NW * _NCH, _CH)
    iidx = ii.reshape(_NW * _NCH, _CH)
    ub = user_bias.reshape(-1)
    ib = item_bias.reshape(-1)

    mesh = plsc.VectorSubcoreMesh(core_axis_name="c", subcore_axis_name="s")
    sc_run = pl.kernel(
        _sc_body,
        out_type=[
            jax.ShapeDtypeStruct((_B, _L), jnp.float32),
            jax.ShapeDtypeStruct((_B,), jnp.float32),
            jax.ShapeDtypeStruct((_B,), jnp.float32),
        ],
        mesh=mesh,
        compiler_params=pltpu.CompilerParams(use_tc_tiling_on_sc=False),
        scratch_types=[
            pltpu.VMEM((_NCH, _CH), jnp.int32),
            pltpu.VMEM((_NCH, _CH), jnp.int32),
            pltpu.VMEM((_CH, _D), jnp.float32),
            pltpu.VMEM((_CH, _D), jnp.float32),
            pltpu.VMEM((_CH, _D), jnp.float32),
            pltpu.VMEM((_CH, _D), jnp.float32),
            pltpu.VMEM((_BPW,), jnp.float32),
            pltpu.VMEM((_BPW,), jnp.float32),
            pltpu.VMEM((_BPW, _L), jnp.float32),
            pltpu.SemaphoreType.DMA,
            pltpu.SemaphoreType.DMA,
        ],
    )
    acc, ubg, ibg = sc_run(uidx, iidx, user_embedding, item_embedding, ub, ib)

    rows_per_blk = 1024
    grid = (_B // rows_per_blk,)
    out = pl.pallas_call(
        _tc_body,
        grid=grid,
        in_specs=[
            pl.BlockSpec((rows_per_blk, _L), lambda i: (i, 0)),
            pl.BlockSpec((rows_per_blk,), lambda i: (i,)),
            pl.BlockSpec((rows_per_blk,), lambda i: (i,)),
        ],
        out_specs=pl.BlockSpec((rows_per_blk,), lambda i: (i,)),
        out_shape=jax.ShapeDtypeStruct((_B,), jnp.float32),
    )(acc, ubg, ibg)
    return out
